# baseline (speedup 1.0000x reference)
import functools

import jax
import jax.numpy as jnp
from jax.experimental import pallas as pl
from jax.experimental.pallas import tpu as pltpu

_K = 1024
_D = 64
_CC = 0.25
_BLK = 2304


def _vq_kernel(x_ref, w_ref, enc_ref, q_ref, loss_ref, perp_ref,
               lacc_ref, cacc_ref, wn_ref, *, n_rows):
    i = pl.program_id(0)
    nsteps = pl.num_programs(0)

    x = x_ref[...].reshape(-1, _D)
    w = w_ref[...]

    @pl.when(i == 0)
    def _prep():
        wn_ref[...] = w / jnp.maximum(
            jnp.sqrt(jnp.sum(w * w, axis=1, keepdims=True)), 1e-12)

    zn = x / jnp.maximum(jnp.sqrt(jnp.sum(x * x, axis=1, keepdims=True)), 1e-12)

    scores = jax.lax.dot_general(zn, wn_ref[...], (((1,), (1,)), ((), ())),
                                 preferred_element_type=jnp.float32)

    m = jnp.max(scores, axis=1, keepdims=True)
    onehot = (scores >= m).astype(jnp.float32)
    enc_ref[...] = onehot

    q = jax.lax.dot_general(onehot, w, (((1,), (0,)), ((), ())),
                            preferred_element_type=jnp.float32)
    q_ref[...] = q.reshape(q_ref.shape)

    d = q - x
    ones = jnp.ones((1, onehot.shape[0]), dtype=jnp.float32)
    part_loss = jax.lax.dot_general(ones, d * d, (((1,), (0,)), ((), ())),
                                    preferred_element_type=jnp.float32)
    part_cnt = jnp.sum(onehot, axis=0, keepdims=True)

    @pl.when(i == 0)
    def _init():
        lacc_ref[...] = jnp.zeros_like(lacc_ref)
        cacc_ref[...] = jnp.zeros_like(cacc_ref)

    new_l = lacc_ref[...] + part_loss
    lacc_ref[...] = new_l
    new_c = cacc_ref[...] + part_cnt
    cacc_ref[...] = new_c

    @pl.when(i == nsteps - 1)
    def _finish():
        loss_ref[...] = jnp.full((1, 1),
                                 (1.0 + _CC) * jnp.sum(new_l) / (n_rows * _D),
                                 dtype=jnp.float32)
        p = new_c / n_rows
        perp_ref[...] = jnp.exp(-jnp.sum(p * jnp.log(p + 1e-10),
                                         keepdims=True))


def kernel(inputs, weight):
    b, t, _ = inputs.shape
    n = b * t
    bblk = _BLK // t
    grid = b // bblk
    enc, q, loss, perp = pl.pallas_call(
        functools.partial(_vq_kernel, n_rows=n),
        grid=(grid,),
        in_specs=[
            pl.BlockSpec((bblk, t, _D), lambda i: (i, 0, 0)),
            pl.BlockSpec((_K, _D), lambda i: (0, 0)),
        ],
        out_specs=[
            pl.BlockSpec((_BLK, _K), lambda i: (i, 0)),
            pl.BlockSpec((bblk, t, _D), lambda i: (i, 0, 0)),
            pl.BlockSpec((1, 1), lambda i: (0, 0)),
            pl.BlockSpec((1, 1), lambda i: (0, 0)),
        ],
        out_shape=[
            jax.ShapeDtypeStruct((n, _K), jnp.float32),
            jax.ShapeDtypeStruct((b, t, _D), jnp.float32),
            jax.ShapeDtypeStruct((1, 1), jnp.float32),
            jax.ShapeDtypeStruct((1, 1), jnp.float32),
        ],
        scratch_shapes=[
            pltpu.VMEM((1, _D), jnp.float32),
            pltpu.VMEM((1, _K), jnp.float32),
            pltpu.VMEM((_K, _D), jnp.float32),
        ],
    )(inputs, weight)
    return (loss[0, 0], q, perp[0, 0], enc)


# speedup vs baseline: 1.3558x; 1.3558x over previous
import functools

import jax
import jax.numpy as jnp
from jax.experimental import pallas as pl
from jax.experimental.pallas import tpu as pltpu

_K = 1024
_D = 64
_CC = 0.25
_BLK = 2304


def _vq_kernel(xt_ref, wt_ref, enc_ref, qt_ref, loss_ref, perp_ref,
               lacc_ref, cacc_ref, wn_ref, w_ref, *, n_rows):
    i = pl.program_id(0)
    nsteps = pl.num_programs(0)

    x = jnp.transpose(xt_ref[...], (0, 2, 1)).reshape(-1, _D)

    @pl.when(i == 0)
    def _prep():
        w0 = jnp.transpose(wt_ref[...], (1, 0))
        w_ref[...] = w0
        wn_ref[...] = w0 / jnp.maximum(
            jnp.sqrt(jnp.sum(w0 * w0, axis=1, keepdims=True)), 1e-12)

    w = w_ref[...]

    zn = x / jnp.maximum(jnp.sqrt(jnp.sum(x * x, axis=1, keepdims=True)), 1e-12)

    scores = jax.lax.dot_general(zn, wn_ref[...], (((1,), (1,)), ((), ())),
                                 preferred_element_type=jnp.float32)

    m = jnp.max(scores, axis=1, keepdims=True)
    onehot = (scores >= m).astype(jnp.float32)
    enc_ref[...] = onehot

    q = jax.lax.dot_general(onehot, w, (((1,), (0,)), ((), ())),
                            preferred_element_type=jnp.float32)
    qt_ref[...] = jnp.transpose(q.reshape(qt_ref.shape[0], -1, _D), (0, 2, 1))

    d = q - x
    ones = jnp.ones((1, onehot.shape[0]), dtype=jnp.float32)
    part_loss = jax.lax.dot_general(ones, d * d, (((1,), (0,)), ((), ())),
                                    preferred_element_type=jnp.float32)
    part_cnt = jnp.sum(onehot, axis=0, keepdims=True)

    @pl.when(i == 0)
    def _init():
        lacc_ref[...] = jnp.zeros_like(lacc_ref)
        cacc_ref[...] = jnp.zeros_like(cacc_ref)

    new_l = lacc_ref[...] + part_loss
    lacc_ref[...] = new_l
    new_c = cacc_ref[...] + part_cnt
    cacc_ref[...] = new_c

    @pl.when(i == nsteps - 1)
    def _finish():
        loss_ref[...] = jnp.full((1, 1),
                                 (1.0 + _CC) * jnp.sum(new_l) / (n_rows * _D),
                                 dtype=jnp.float32)
        p = new_c / n_rows
        perp_ref[...] = jnp.exp(-jnp.sum(p * jnp.log(p + 1e-10),
                                         keepdims=True))


def kernel(inputs, weight):
    b, t, _ = inputs.shape
    n = b * t
    bblk = _BLK // t
    grid = b // bblk
    xt = jnp.swapaxes(inputs, 1, 2)
    wt = jnp.swapaxes(weight, 0, 1)
    enc, qt, loss, perp = pl.pallas_call(
        functools.partial(_vq_kernel, n_rows=n),
        grid=(grid,),
        in_specs=[
            pl.BlockSpec((bblk, _D, t), lambda i: (i, 0, 0)),
            pl.BlockSpec((_D, _K), lambda i: (0, 0)),
        ],
        out_specs=[
            pl.BlockSpec((_BLK, _K), lambda i: (i, 0)),
            pl.BlockSpec((bblk, _D, t), lambda i: (i, 0, 0)),
            pl.BlockSpec((1, 1), lambda i: (0, 0)),
            pl.BlockSpec((1, 1), lambda i: (0, 0)),
        ],
        out_shape=[
            jax.ShapeDtypeStruct((n, _K), jnp.float32),
            jax.ShapeDtypeStruct((b, _D, t), jnp.float32),
            jax.ShapeDtypeStruct((1, 1), jnp.float32),
            jax.ShapeDtypeStruct((1, 1), jnp.float32),
        ],
        scratch_shapes=[
            pltpu.VMEM((1, _D), jnp.float32),
            pltpu.VMEM((1, _K), jnp.float32),
            pltpu.VMEM((_K, _D), jnp.float32),
            pltpu.VMEM((_K, _D), jnp.float32),
        ],
    )(xt, wt)
    return (loss[0, 0], jnp.swapaxes(qt, 1, 2), perp[0, 0], enc)
